# baseline (speedup 1.0000x reference)
import functools

import jax
import jax.numpy as jnp
from jax import lax
from jax.experimental import pallas as pl
from jax.experimental.pallas import tpu as pltpu
from jax.experimental.pallas import tpu_sc as plsc

N = 10000
E = 320000
H = 128
NC = 2
NS = 16
NW = NC * NS
CHUNK = 128
NCH = 4 * (-(-E // (NW * CHUNK * 4)))
EPW = NCH * CHUNK
EPAD = EPW * NW
ACC_ROWS = 10240
ZB = 64
BROWS = 2000


RS = 8


def _sc_agg_body(h_hbm, pidx_hbm, out_hbm, pk_v, gring, sring, rows_a, rows_b,
                 acc, sem, gsem_a, gsem_b, ssem_a, ssem_b):
    c = lax.axis_index("c")
    s = lax.axis_index("s")
    wid = s * NC + c

    idx_cp = pltpu.async_copy(pidx_hbm.at[wid], pk_v, sem)
    zz = jnp.zeros((16,), jnp.float32)

    def zfill(r, carry):
        for j in range(H // 16):
            rows_a[r, pl.ds(j * 16, 16)] = zz
        return carry

    lax.fori_loop(0, CHUNK, zfill, 0)
    zrows = ACC_ROWS // NS

    def zcopy(r, carry):
        pltpu.sync_copy(rows_a, acc.at[pl.ds(s * zrows + r * CHUNK, CHUNK)])
        return carry

    lax.fori_loop(0, zrows // CHUNK, zcopy, 0)
    idx_cp.wait()
    plsc.subcore_barrier()

    def unpack(j):
        r = j % RS
        for t in range(H // 16):
            pk = pk_v[j, pl.ds(t * 16, 16)]
            gring[r, pl.ds(t * 16, 16)] = pk & 0xFFFF
            sring[r, pl.ds(t * 16, 16)] = pk >> 16

    bufs = ((rows_a, gsem_a, ssem_a), (rows_b, gsem_b, ssem_b))
    unpack(0)
    pltpu.async_copy(h_hbm.at[gring.at[0]], rows_a, gsem_a)
    unpack(1)
    pltpu.async_copy(h_hbm.at[gring.at[1]], rows_b, gsem_b)
    pltpu.make_async_copy(h_hbm.at[gring.at[0]], rows_a, gsem_a).wait()
    pltpu.async_copy(rows_a, acc.at[sring.at[0]], ssem_a, add=True)

    def body(o, carry):
        for par in range(2):
            i = 2 * o + par + 2
            buf, gsem, ssem = bufs[par]
            obuf, ogsem, ossem = bufs[1 - par]
            pltpu.make_async_copy(h_hbm.at[gring.at[(i - 1) % RS]], obuf,
                                  ogsem).wait()
            pltpu.async_copy(obuf, acc.at[sring.at[(i - 1) % RS]], ossem,
                             add=True)
            pltpu.make_async_copy(buf, acc.at[sring.at[(i - 2) % RS]],
                                  ssem).wait()
            unpack(i)
            pltpu.async_copy(h_hbm.at[gring.at[i % RS]], buf, gsem)
        return carry

    lax.fori_loop(0, (NCH - 2) // 2, body, 0)
    bufl, gseml, sseml = bufs[(NCH - 1) % 2]
    bufp, gsemp, ssemp = bufs[NCH % 2]
    pltpu.make_async_copy(h_hbm.at[gring.at[(NCH - 1) % RS]], bufl,
                          gseml).wait()
    pltpu.async_copy(bufl, acc.at[sring.at[(NCH - 1) % RS]], sseml, add=True)
    pltpu.make_async_copy(bufp, acc.at[sring.at[0]], ssemp).wait()
    pltpu.make_async_copy(bufl, acc.at[sring.at[0]], sseml).wait()
    plsc.subcore_barrier()

    orows = 624
    pltpu.sync_copy(acc.at[pl.ds(s * orows, orows)],
                    out_hbm.at[c, pl.ds(s * orows, orows)])

    @pl.when(s == 0)
    def _():
        pltpu.sync_copy(acc.at[pl.ds(NS * orows, N - NS * orows)],
                        out_hbm.at[c, pl.ds(NS * orows, N - NS * orows)])


@functools.cache
def _make_sc_agg():
    mesh = plsc.VectorSubcoreMesh(core_axis_name="c", subcore_axis_name="s")
    return pl.kernel(
        _sc_agg_body,
        out_type=jax.ShapeDtypeStruct((NC, N, H), jnp.float32),
        mesh=mesh,
        scratch_types=[
            pltpu.VMEM((NCH, CHUNK), jnp.int32),
            pltpu.VMEM((RS, CHUNK), jnp.int32),
            pltpu.VMEM((RS, CHUNK), jnp.int32),
            pltpu.VMEM((CHUNK, H), jnp.float32),
            pltpu.VMEM((CHUNK, H), jnp.float32),
            pltpu.VMEM_SHARED((ACC_ROWS, H), jnp.float32),
        ] + [pltpu.SemaphoreType.DMA] * 5,
    )


def _agg(h, pidx):
    return _make_sc_agg()(h, pidx)


def _ln(t, g, b):
    m = jnp.mean(t, axis=-1, keepdims=True)
    d = t - m
    v = jnp.mean(d * d, axis=-1, keepdims=True)
    return d * lax.rsqrt(v + 1e-5) * g + b


def _gin_body(eps_ref, x_ref, p_ref, w0_ref, g0_ref, b0_ref, w1_ref, dir_ref,
              g2_ref, b2_ref, o_ref):
    y = x_ref[...] * eps_ref[0, 0] + p_ref[0] + p_ref[1]
    t = jnp.dot(y, w0_ref[...], preferred_element_type=jnp.float32)
    t = jnp.maximum(_ln(t, g0_ref[...], b0_ref[...]), 0.0)
    z = jnp.dot(t, w1_ref[...], preferred_element_type=jnp.float32)
    z = jnp.maximum(z + dir_ref[...], 0.0)
    o_ref[...] = _ln(z, g2_ref[...], b2_ref[...])


def _tc_gin(x, p, eps1, w0, g0, b0, w1, dirv, g2, b2):
    grid = (N // BROWS,)
    row = pl.BlockSpec((BROWS, H), lambda i: (i, 0))
    full = pl.BlockSpec((H, H), lambda i: (0, 0))
    vec = pl.BlockSpec((1, H), lambda i: (0, 0))
    return pl.pallas_call(
        _gin_body,
        grid=grid,
        in_specs=[
            pl.BlockSpec(memory_space=pltpu.SMEM),
            row,
            pl.BlockSpec((NC, BROWS, H), lambda i: (0, i, 0)),
            full, vec, vec, full, vec, vec, vec,
        ],
        out_specs=row,
        out_shape=jax.ShapeDtypeStruct((N, H), jnp.float32),
    )(eps1, x, p, w0, g0, b0, w1, dirv, g2, b2)


def _mlp2_body(x_ref, w0_ref, g_ref, b_ref, w1_ref, o_ref, *, inner):
    t = jnp.dot(x_ref[...], w0_ref[...], preferred_element_type=jnp.float32)
    if inner == H:
        t = _ln(t, g_ref[...], b_ref[...])
    else:
        m = jnp.sum(t, axis=-1, keepdims=True) / inner
        d = t - m
        mask = lax.broadcasted_iota(jnp.int32, t.shape, 1) < inner
        v = jnp.sum(jnp.where(mask, d * d, 0.0), axis=-1, keepdims=True) / inner
        t = d * lax.rsqrt(v + 1e-5) * g_ref[...] + b_ref[...]
    t = jnp.maximum(t, 0.0)
    o_ref[...] = jnp.dot(t, w1_ref[...], preferred_element_type=jnp.float32)


def _tc_mlp2(x, w0, g, b, w1, inner):
    grid = (N // BROWS,)
    row = pl.BlockSpec((BROWS, H), lambda i: (i, 0))
    full = pl.BlockSpec((H, H), lambda i: (0, 0))
    vec = pl.BlockSpec((1, H), lambda i: (0, 0))
    return pl.pallas_call(
        functools.partial(_mlp2_body, inner=inner),
        grid=grid,
        in_specs=[row, full, vec, vec, full],
        out_specs=row,
        out_shape=jax.ShapeDtypeStruct((N, H), jnp.float32),
    )(x, w0, g, b, w1)


def _pad_mlp2(p, inner):
    w0, w1 = p["Ws"]
    g, b = p["ln_g"][0], p["ln_b"][0]
    if inner != H:
        w0 = jnp.pad(w0, ((0, 0), (0, H - inner)))
        w1 = jnp.pad(w1, ((0, H - inner), (0, 0)))
        g = jnp.pad(g, (0, H - inner))
        b = jnp.pad(b, (0, H - inner))
    return w0, g.reshape(1, H), b.reshape(1, H), w1


def _gin_step(h, p_parts, mp, eps, dirv, g2, b2):
    w0, w1 = mp["Ws"]
    g0 = mp["ln_g"][0].reshape(1, H)
    b0 = mp["ln_b"][0].reshape(1, H)
    eps1 = (1.0 + eps).reshape(1, 1).astype(jnp.float32)
    return _tc_gin(h, p_parts, eps1, w0, g0, b0, w1, dirv,
                   g2.reshape(1, H), b2.reshape(1, H))


def kernel(x, edge_index, params):
    src = edge_index[0]
    dst = edge_index[1]
    npad = EPAD - E
    pad_g = jnp.arange(npad, dtype=jnp.int32) % N
    pad_s = N + (jnp.arange(npad, dtype=jnp.int32) % (ACC_ROWS - N))
    gd = jnp.concatenate([src, pad_g])
    sd = jnp.concatenate([dst, pad_s])
    gu = jnp.concatenate([dst, pad_g])
    su = jnp.concatenate([src, pad_s])
    p_down = (gd | (sd << 16)).reshape(NW, NCH, CHUNK)
    p_up = (gu | (su << 16)).reshape(NW, NCH, CHUNK)

    def downup(h, lp):
        p = _agg(h, p_down)
        h = _gin_step(h, p, lp["down"], lp["down_eps"], lp["dir_emb"][0:1],
                      lp["ln1_g"], lp["ln1_b"])
        p = _agg(h, p_up)
        return _gin_step(h, p, lp["up"], lp["up_eps"], lp["dir_emb"][1:2],
                         lp["ln2_g"], lp["ln2_b"])

    h = _tc_mlp2(x, *_pad_mlp2(params["enc_in_proj"], H), H)
    for lp in params["enc_layers"]:
        h = downup(h, lp)
    h = _tc_mlp2(h, *_pad_mlp2(params["dec_in_proj"], H), H)
    for lp in params["dec_layers"]:
        h = downup(h, lp)
    return _tc_mlp2(h, *_pad_mlp2(params["dec_out_proj"], 16), 16)


# speedup vs baseline: 1.2125x; 1.2125x over previous
import functools

import jax
import jax.numpy as jnp
from jax import lax
from jax.experimental import pallas as pl
from jax.experimental.pallas import tpu as pltpu
from jax.experimental.pallas import tpu_sc as plsc

N = 10000
E = 320000
H = 128
NC = 2
NS = 16
NW = NC * NS
CHUNK = 128
NCH = 4 * (-(-E // (NW * CHUNK * 4)))
EPW = NCH * CHUNK
EPAD = EPW * NW
ACC_ROWS = 10240
ZB = 64
BROWS = 2000


RS = 8


def _sc_agg_body(h_hbm, pidx_hbm, out_hbm, pk_v, gring, sring, rows_a, rows_b,
                 acc, sem, gsem_a, gsem_b):
    c = lax.axis_index("c")
    s = lax.axis_index("s")
    wid = s * NC + c

    idx_cp = pltpu.async_copy(pidx_hbm.at[wid], pk_v, sem)
    zz = jnp.zeros((16,), jnp.float32)

    def zfill(r, carry):
        for j in range(H // 16):
            rows_a[r, pl.ds(j * 16, 16)] = zz
        return carry

    lax.fori_loop(0, CHUNK, zfill, 0)
    zrows = ACC_ROWS // NS

    def zcopy(r, carry):
        pltpu.sync_copy(rows_a, acc.at[pl.ds(s * zrows + r * CHUNK, CHUNK)])
        return carry

    lax.fori_loop(0, zrows // CHUNK, zcopy, 0)
    idx_cp.wait()
    plsc.subcore_barrier()

    def unpack(j):
        r = j % RS
        for t in range(H // 16):
            pk = pk_v[j, pl.ds(t * 16, 16)]
            gring[r, pl.ds(t * 16, 16)] = pk & 0xFFFF
            sring[r, pl.ds(t * 16, 16)] = pk >> 16

    unpack(0)
    pltpu.async_copy(h_hbm.at[gring.at[0]], rows_a, gsem_a)
    unpack(1)
    pltpu.async_copy(h_hbm.at[gring.at[1]], rows_b, gsem_b)

    bufs = ((rows_a, gsem_a), (rows_b, gsem_b))

    def body(o, carry):
        for par in range(2):
            i = 2 * o + par
            buf, gsem = bufs[par]
            pltpu.make_async_copy(h_hbm.at[gring.at[i % RS]], buf, gsem).wait()
            pltpu.sync_copy(buf, acc.at[sring.at[i % RS]], add=True)
            unpack(i + 2)
            pltpu.async_copy(h_hbm.at[gring.at[(i + 2) % RS]], buf, gsem)
        return carry

    lax.fori_loop(0, (NCH - 2) // 2, body, 0)
    for i in (NCH - 2, NCH - 1):
        buf, gsem = bufs[i % 2]
        pltpu.make_async_copy(h_hbm.at[gring.at[i % RS]], buf, gsem).wait()
        pltpu.sync_copy(buf, acc.at[sring.at[i % RS]], add=True)
    plsc.subcore_barrier()

    orows = 624
    pltpu.sync_copy(acc.at[pl.ds(s * orows, orows)],
                    out_hbm.at[c, pl.ds(s * orows, orows)])

    @pl.when(s == 0)
    def _():
        pltpu.sync_copy(acc.at[pl.ds(NS * orows, N - NS * orows)],
                        out_hbm.at[c, pl.ds(NS * orows, N - NS * orows)])


@functools.cache
def _make_sc_agg():
    mesh = plsc.VectorSubcoreMesh(core_axis_name="c", subcore_axis_name="s")
    return pl.kernel(
        _sc_agg_body,
        out_type=jax.ShapeDtypeStruct((NC, N, H), jnp.float32),
        mesh=mesh,
        scratch_types=[
            pltpu.VMEM((NCH, CHUNK), jnp.int32),
            pltpu.VMEM((RS, CHUNK), jnp.int32),
            pltpu.VMEM((RS, CHUNK), jnp.int32),
            pltpu.VMEM((CHUNK, H), jnp.float32),
            pltpu.VMEM((CHUNK, H), jnp.float32),
            pltpu.VMEM_SHARED((ACC_ROWS, H), jnp.float32),
        ] + [pltpu.SemaphoreType.DMA] * 3,
    )


def _agg(h, pidx):
    return _make_sc_agg()(h, pidx)


def _ln(t, g, b):
    m = jnp.mean(t, axis=-1, keepdims=True)
    d = t - m
    v = jnp.mean(d * d, axis=-1, keepdims=True)
    return d * lax.rsqrt(v + 1e-5) * g + b


def _gin_body(eps_ref, x_ref, p_ref, w0_ref, g0_ref, b0_ref, w1_ref, dir_ref,
              g2_ref, b2_ref, o_ref):
    y = x_ref[...] * eps_ref[0, 0] + p_ref[0] + p_ref[1]
    t = jnp.dot(y, w0_ref[...], preferred_element_type=jnp.float32)
    t = jnp.maximum(_ln(t, g0_ref[...], b0_ref[...]), 0.0)
    z = jnp.dot(t, w1_ref[...], preferred_element_type=jnp.float32)
    z = jnp.maximum(z + dir_ref[...], 0.0)
    o_ref[...] = _ln(z, g2_ref[...], b2_ref[...])


def _tc_gin(x, p, eps1, w0, g0, b0, w1, dirv, g2, b2):
    grid = (N // BROWS,)
    row = pl.BlockSpec((BROWS, H), lambda i: (i, 0))
    full = pl.BlockSpec((H, H), lambda i: (0, 0))
    vec = pl.BlockSpec((1, H), lambda i: (0, 0))
    return pl.pallas_call(
        _gin_body,
        grid=grid,
        in_specs=[
            pl.BlockSpec(memory_space=pltpu.SMEM),
            row,
            pl.BlockSpec((NC, BROWS, H), lambda i: (0, i, 0)),
            full, vec, vec, full, vec, vec, vec,
        ],
        out_specs=row,
        out_shape=jax.ShapeDtypeStruct((N, H), jnp.float32),
    )(eps1, x, p, w0, g0, b0, w1, dirv, g2, b2)


def _mlp2_body(x_ref, w0_ref, g_ref, b_ref, w1_ref, o_ref, *, inner):
    t = jnp.dot(x_ref[...], w0_ref[...], preferred_element_type=jnp.float32)
    if inner == H:
        t = _ln(t, g_ref[...], b_ref[...])
    else:
        m = jnp.sum(t, axis=-1, keepdims=True) / inner
        d = t - m
        mask = lax.broadcasted_iota(jnp.int32, t.shape, 1) < inner
        v = jnp.sum(jnp.where(mask, d * d, 0.0), axis=-1, keepdims=True) / inner
        t = d * lax.rsqrt(v + 1e-5) * g_ref[...] + b_ref[...]
    t = jnp.maximum(t, 0.0)
    o_ref[...] = jnp.dot(t, w1_ref[...], preferred_element_type=jnp.float32)


def _tc_mlp2(x, w0, g, b, w1, inner):
    grid = (N // BROWS,)
    row = pl.BlockSpec((BROWS, H), lambda i: (i, 0))
    full = pl.BlockSpec((H, H), lambda i: (0, 0))
    vec = pl.BlockSpec((1, H), lambda i: (0, 0))
    return pl.pallas_call(
        functools.partial(_mlp2_body, inner=inner),
        grid=grid,
        in_specs=[row, full, vec, vec, full],
        out_specs=row,
        out_shape=jax.ShapeDtypeStruct((N, H), jnp.float32),
    )(x, w0, g, b, w1)


def _pad_mlp2(p, inner):
    w0, w1 = p["Ws"]
    g, b = p["ln_g"][0], p["ln_b"][0]
    if inner != H:
        w0 = jnp.pad(w0, ((0, 0), (0, H - inner)))
        w1 = jnp.pad(w1, ((0, H - inner), (0, 0)))
        g = jnp.pad(g, (0, H - inner))
        b = jnp.pad(b, (0, H - inner))
    return w0, g.reshape(1, H), b.reshape(1, H), w1


def _gin_step(h, p_parts, mp, eps, dirv, g2, b2):
    w0, w1 = mp["Ws"]
    g0 = mp["ln_g"][0].reshape(1, H)
    b0 = mp["ln_b"][0].reshape(1, H)
    eps1 = (1.0 + eps).reshape(1, 1).astype(jnp.float32)
    return _tc_gin(h, p_parts, eps1, w0, g0, b0, w1, dirv,
                   g2.reshape(1, H), b2.reshape(1, H))


def kernel(x, edge_index, params):
    src = edge_index[0]
    dst = edge_index[1]
    npad = EPAD - E
    pad_g = jnp.arange(npad, dtype=jnp.int32) % N
    pad_s = N + (jnp.arange(npad, dtype=jnp.int32) % (ACC_ROWS - N))
    gd = jnp.concatenate([src, pad_g])
    sd = jnp.concatenate([dst, pad_s])
    gu = jnp.concatenate([dst, pad_g])
    su = jnp.concatenate([src, pad_s])
    p_down = (gd | (sd << 16)).reshape(NW, NCH, CHUNK)
    p_up = (gu | (su << 16)).reshape(NW, NCH, CHUNK)

    def downup(h, lp):
        p = _agg(h, p_down)
        h = _gin_step(h, p, lp["down"], lp["down_eps"], lp["dir_emb"][0:1],
                      lp["ln1_g"], lp["ln1_b"])
        p = _agg(h, p_up)
        return _gin_step(h, p, lp["up"], lp["up_eps"], lp["dir_emb"][1:2],
                         lp["ln2_g"], lp["ln2_b"])

    h = _tc_mlp2(x, *_pad_mlp2(params["enc_in_proj"], H), H)
    for lp in params["enc_layers"]:
        h = downup(h, lp)
    h = _tc_mlp2(h, *_pad_mlp2(params["dec_in_proj"], H), H)
    for lp in params["dec_layers"]:
        h = downup(h, lp)
    return _tc_mlp2(h, *_pad_mlp2(params["dec_out_proj"], 16), 16)


# speedup vs baseline: 1.2164x; 1.0032x over previous
import functools

import jax
import jax.numpy as jnp
from jax import lax
from jax.experimental import pallas as pl
from jax.experimental.pallas import tpu as pltpu
from jax.experimental.pallas import tpu_sc as plsc

N = 10000
E = 320000
H = 128
NC = 2
NS = 16
NW = NC * NS
CHUNK = 128
NCH = 4 * (-(-E // (NW * CHUNK * 4)))
EPW = NCH * CHUNK
EPAD = EPW * NW
ACC_ROWS = 10240
ZB = 64
BROWS = 2000


RS = 8


def _sc_agg_body(h_hbm, pidx_hbm, out_hbm, pk_v, gring, sring, rows_a, rows_b,
                 acc, sem, gsem_a, gsem_b):
    c = lax.axis_index("c")
    s = lax.axis_index("s")
    wid = s * NC + c

    idx_cp = pltpu.async_copy(pidx_hbm.at[wid], pk_v, sem)
    zz = jnp.zeros((16,), jnp.float32)

    def zfill(r, carry):
        for j in range(H // 16):
            rows_a[r, pl.ds(j * 16, 16)] = zz
        return carry

    lax.fori_loop(0, CHUNK, zfill, 0)
    zrows = ACC_ROWS // NS

    def zcopy(r, carry):
        pltpu.sync_copy(rows_a, acc.at[pl.ds(s * zrows + r * CHUNK, CHUNK)])
        return carry

    lax.fori_loop(0, zrows // CHUNK, zcopy, 0)
    idx_cp.wait()

    def unpack(j):
        r = j % RS
        for t in range(H // 16):
            pk = pk_v[j, pl.ds(t * 16, 16)]
            gring[r, pl.ds(t * 16, 16)] = pk & 0xFFFF
            sring[r, pl.ds(t * 16, 16)] = pk >> 16

    unpack(0)
    pltpu.async_copy(h_hbm.at[gring.at[0]], rows_a, gsem_a)
    unpack(1)
    pltpu.async_copy(h_hbm.at[gring.at[1]], rows_b, gsem_b)
    plsc.subcore_barrier()

    bufs = ((rows_a, gsem_a), (rows_b, gsem_b))

    def body(o, carry):
        for par in range(2):
            i = 2 * o + par
            buf, gsem = bufs[par]
            pltpu.make_async_copy(h_hbm.at[gring.at[i % RS]], buf, gsem).wait()
            pltpu.sync_copy(buf, acc.at[sring.at[i % RS]], add=True)
            unpack(i + 2)
            pltpu.async_copy(h_hbm.at[gring.at[(i + 2) % RS]], buf, gsem)
        return carry

    lax.fori_loop(0, (NCH - 2) // 2, body, 0)
    for i in (NCH - 2, NCH - 1):
        buf, gsem = bufs[i % 2]
        pltpu.make_async_copy(h_hbm.at[gring.at[i % RS]], buf, gsem).wait()
        pltpu.sync_copy(buf, acc.at[sring.at[i % RS]], add=True)
    plsc.subcore_barrier()

    orows = 624
    pltpu.sync_copy(acc.at[pl.ds(s * orows, orows)],
                    out_hbm.at[c, pl.ds(s * orows, orows)])

    @pl.when(s == 0)
    def _():
        pltpu.sync_copy(acc.at[pl.ds(NS * orows, N - NS * orows)],
                        out_hbm.at[c, pl.ds(NS * orows, N - NS * orows)])


@functools.cache
def _make_sc_agg():
    mesh = plsc.VectorSubcoreMesh(core_axis_name="c", subcore_axis_name="s")
    return pl.kernel(
        _sc_agg_body,
        out_type=jax.ShapeDtypeStruct((NC, N, H), jnp.float32),
        mesh=mesh,
        scratch_types=[
            pltpu.VMEM((NCH, CHUNK), jnp.int32),
            pltpu.VMEM((RS, CHUNK), jnp.int32),
            pltpu.VMEM((RS, CHUNK), jnp.int32),
            pltpu.VMEM((CHUNK, H), jnp.float32),
            pltpu.VMEM((CHUNK, H), jnp.float32),
            pltpu.VMEM_SHARED((ACC_ROWS, H), jnp.float32),
        ] + [pltpu.SemaphoreType.DMA] * 3,
    )


def _agg(h, pidx):
    return _make_sc_agg()(h, pidx)


def _ln(t, g, b):
    m = jnp.mean(t, axis=-1, keepdims=True)
    d = t - m
    v = jnp.mean(d * d, axis=-1, keepdims=True)
    return d * lax.rsqrt(v + 1e-5) * g + b


def _gin_body(eps_ref, x_ref, p_ref, w0_ref, g0_ref, b0_ref, w1_ref, dir_ref,
              g2_ref, b2_ref, o_ref):
    y = x_ref[...] * eps_ref[0, 0] + p_ref[0] + p_ref[1]
    t = jnp.dot(y, w0_ref[...], preferred_element_type=jnp.float32)
    t = jnp.maximum(_ln(t, g0_ref[...], b0_ref[...]), 0.0)
    z = jnp.dot(t, w1_ref[...], preferred_element_type=jnp.float32)
    z = jnp.maximum(z + dir_ref[...], 0.0)
    o_ref[...] = _ln(z, g2_ref[...], b2_ref[...])


def _tc_gin(x, p, eps1, w0, g0, b0, w1, dirv, g2, b2):
    grid = (N // BROWS,)
    row = pl.BlockSpec((BROWS, H), lambda i: (i, 0))
    full = pl.BlockSpec((H, H), lambda i: (0, 0))
    vec = pl.BlockSpec((1, H), lambda i: (0, 0))
    return pl.pallas_call(
        _gin_body,
        grid=grid,
        in_specs=[
            pl.BlockSpec(memory_space=pltpu.SMEM),
            row,
            pl.BlockSpec((NC, BROWS, H), lambda i: (0, i, 0)),
            full, vec, vec, full, vec, vec, vec,
        ],
        out_specs=row,
        out_shape=jax.ShapeDtypeStruct((N, H), jnp.float32),
    )(eps1, x, p, w0, g0, b0, w1, dirv, g2, b2)


def _mlp2_body(x_ref, w0_ref, g_ref, b_ref, w1_ref, o_ref, *, inner):
    t = jnp.dot(x_ref[...], w0_ref[...], preferred_element_type=jnp.float32)
    if inner == H:
        t = _ln(t, g_ref[...], b_ref[...])
    else:
        m = jnp.sum(t, axis=-1, keepdims=True) / inner
        d = t - m
        mask = lax.broadcasted_iota(jnp.int32, t.shape, 1) < inner
        v = jnp.sum(jnp.where(mask, d * d, 0.0), axis=-1, keepdims=True) / inner
        t = d * lax.rsqrt(v + 1e-5) * g_ref[...] + b_ref[...]
    t = jnp.maximum(t, 0.0)
    o_ref[...] = jnp.dot(t, w1_ref[...], preferred_element_type=jnp.float32)


def _tc_mlp2(x, w0, g, b, w1, inner):
    grid = (N // BROWS,)
    row = pl.BlockSpec((BROWS, H), lambda i: (i, 0))
    full = pl.BlockSpec((H, H), lambda i: (0, 0))
    vec = pl.BlockSpec((1, H), lambda i: (0, 0))
    return pl.pallas_call(
        functools.partial(_mlp2_body, inner=inner),
        grid=grid,
        in_specs=[row, full, vec, vec, full],
        out_specs=row,
        out_shape=jax.ShapeDtypeStruct((N, H), jnp.float32),
    )(x, w0, g, b, w1)


def _pad_mlp2(p, inner):
    w0, w1 = p["Ws"]
    g, b = p["ln_g"][0], p["ln_b"][0]
    if inner != H:
        w0 = jnp.pad(w0, ((0, 0), (0, H - inner)))
        w1 = jnp.pad(w1, ((0, H - inner), (0, 0)))
        g = jnp.pad(g, (0, H - inner))
        b = jnp.pad(b, (0, H - inner))
    return w0, g.reshape(1, H), b.reshape(1, H), w1


def _gin_step(h, p_parts, mp, eps, dirv, g2, b2):
    w0, w1 = mp["Ws"]
    g0 = mp["ln_g"][0].reshape(1, H)
    b0 = mp["ln_b"][0].reshape(1, H)
    eps1 = (1.0 + eps).reshape(1, 1).astype(jnp.float32)
    return _tc_gin(h, p_parts, eps1, w0, g0, b0, w1, dirv,
                   g2.reshape(1, H), b2.reshape(1, H))


def kernel(x, edge_index, params):
    src = edge_index[0]
    dst = edge_index[1]
    npad = EPAD - E
    pad_g = jnp.arange(npad, dtype=jnp.int32) % N
    pad_s = N + (jnp.arange(npad, dtype=jnp.int32) % (ACC_ROWS - N))
    gd = jnp.concatenate([src, pad_g])
    sd = jnp.concatenate([dst, pad_s])
    gu = jnp.concatenate([dst, pad_g])
    su = jnp.concatenate([src, pad_s])
    p_down = (gd | (sd << 16)).reshape(NW, NCH, CHUNK)
    p_up = (gu | (su << 16)).reshape(NW, NCH, CHUNK)

    def downup(h, lp):
        p = _agg(h, p_down)
        h = _gin_step(h, p, lp["down"], lp["down_eps"], lp["dir_emb"][0:1],
                      lp["ln1_g"], lp["ln1_b"])
        p = _agg(h, p_up)
        return _gin_step(h, p, lp["up"], lp["up_eps"], lp["dir_emb"][1:2],
                         lp["ln2_g"], lp["ln2_b"])

    h = _tc_mlp2(x, *_pad_mlp2(params["enc_in_proj"], H), H)
    for lp in params["enc_layers"]:
        h = downup(h, lp)
    h = _tc_mlp2(h, *_pad_mlp2(params["dec_in_proj"], H), H)
    for lp in params["dec_layers"]:
        h = downup(h, lp)
    return _tc_mlp2(h, *_pad_mlp2(params["dec_out_proj"], 16), 16)
